# Initial kernel scaffold; baseline (speedup 1.0000x reference)
import jax
import jax.numpy as jnp
from jax.experimental import pallas as pl


def kernel(embedding, batch_size, embedding_coreset):
    raise NotImplementedError("write your pallas kernel here")


# speedup vs baseline: 1.1660x; 1.1660x over previous
import jax
import jax.numpy as jnp
from jax import lax
from jax.experimental import pallas as pl
from jax.experimental.pallas import tpu as pltpu

_B = 8
_P = 784
_M = 8192
_D = 384
_K = 9
_CHUNK = 2048


def _body(emb_ref, cs_ref, out_ref):
    A = emb_ref[...]
    a2 = jnp.sum(A * A, axis=1, keepdims=True)
    ones_row = jnp.ones((1, _D), jnp.float32)

    run_min = jnp.full((_P, 1), jnp.inf, jnp.float32)
    run_idx = jnp.zeros((_P, 1), jnp.int32)
    iota_c = lax.broadcasted_iota(jnp.int32, (_P, _CHUNK), 1)
    for c in range(_M // _CHUNK):
        Bc = cs_ref[c * _CHUNK:(c + 1) * _CHUNK, :]
        G = lax.dot_general(A, Bc, (((1,), (1,)), ((), ())),
                            preferred_element_type=jnp.float32)
        b2c = lax.dot_general(ones_row, Bc * Bc, (((1,), (1,)), ((), ())),
                              preferred_element_type=jnp.float32)
        d2c = jnp.maximum(a2 + b2c - 2.0 * G, 1e-12)
        m = jnp.min(d2c, axis=1, keepdims=True)
        ic = jnp.min(jnp.where(d2c == m, iota_c, _M),
                     axis=1, keepdims=True) + c * _CHUNK
        upd = m < run_min
        run_idx = jnp.where(upd, ic, run_idx)
        run_min = jnp.where(upd, m, run_min)

    maxv = jnp.max(run_min)
    iota_p = lax.broadcasted_iota(jnp.int32, (_P, 1), 0)
    p_star = jnp.min(jnp.where(run_min == maxv, iota_p, _P))
    sel = iota_p == p_star
    score = jnp.sqrt(maxv)
    nn_idx = jnp.sum(jnp.where(sel, run_idx, 0))

    feats = emb_ref[pl.ds(p_star, 1), :]
    a2p = jnp.sum(feats * feats)
    nn = cs_ref[pl.ds(nn_idx, 1), :]
    nn2 = jnp.sum(nn * nn)

    CS = cs_ref[...]
    Gn = lax.dot_general(nn, CS, (((1,), (1,)), ((), ())),
                         preferred_element_type=jnp.float32)
    b2row = lax.dot_general(ones_row, CS * CS, (((1,), (1,)), ((), ())),
                            preferred_element_type=jnp.float32)
    dv = jnp.sqrt(jnp.maximum(nn2 + b2row - 2.0 * Gn, 1e-12))

    iota_m = lax.broadcasted_iota(jnp.int32, (1, _M), 1)
    dks = []
    for _ in range(_K):
        mv = jnp.min(dv)
        ik = jnp.min(jnp.where(dv == mv, iota_m, _M))
        row = cs_ref[pl.ds(ik, 1), :]
        b2k = jnp.sum(row * row)
        dk = jnp.sqrt(jnp.maximum(a2p + b2k - 2.0 * jnp.sum(feats * row),
                                  1e-12))
        dks.append(dk)
        dv = jnp.where(iota_m == ik, jnp.inf, dv)

    m9 = dks[0]
    for d in dks[1:]:
        m9 = jnp.maximum(m9, d)
    es = [jnp.exp(d - m9) for d in dks]
    tot = es[0]
    for e in es[1:]:
        tot = tot + e
    w = 1.0 - es[0] / tot
    out_ref[...] = jnp.broadcast_to(w * score, (1, 1, 128))


def kernel(embedding, batch_size, embedding_coreset):
    res = pl.pallas_call(
        _body,
        grid=(_B,),
        in_specs=[
            pl.BlockSpec((_P, _D), lambda b: (b, 0)),
            pl.BlockSpec((_M, _D), lambda b: (0, 0)),
        ],
        out_specs=pl.BlockSpec((1, 1, 128), lambda b: (b, 0, 0)),
        out_shape=jax.ShapeDtypeStruct((_B, 1, 128), jnp.float32),
        compiler_params=pltpu.CompilerParams(
            dimension_semantics=("arbitrary",),
        ),
    )(embedding, embedding_coreset)
    return res[:, 0, 0] + 0.0 * batch_size


# speedup vs baseline: 1.8919x; 1.6226x over previous
import jax
import jax.numpy as jnp
from jax import lax
from jax.experimental import pallas as pl
from jax.experimental.pallas import tpu as pltpu

_B = 8
_P = 784
_M = 8192
_D = 384
_K = 9
_CHUNK = 2048


def _dot_nt(x, y):
    return lax.dot_general(x, y, (((1,), (1,)), ((), ())),
                           preferred_element_type=jnp.float32)


def _body(emb_ref, cs_ref, out_ref, b2_ref):
    ones_row = jnp.ones((1, _D), jnp.float32)

    @pl.when(pl.program_id(0) == 0)
    def _():
        for c in range(_M // _CHUNK):
            Bc = cs_ref[c * _CHUNK:(c + 1) * _CHUNK, :]
            b2_ref[:, c * _CHUNK:(c + 1) * _CHUNK] = _dot_nt(ones_row, Bc * Bc)

    A = emb_ref[...]
    a2 = jnp.sum(A * A, axis=1, keepdims=True)

    run = jnp.full((_P, 1), jnp.inf, jnp.float32)
    for c in range(_M // _CHUNK):
        Bc = cs_ref[c * _CHUNK:(c + 1) * _CHUNK, :]
        G = _dot_nt(A, Bc)
        S = b2_ref[:, c * _CHUNK:(c + 1) * _CHUNK] - 2.0 * G
        run = jnp.minimum(run, jnp.min(S, axis=1, keepdims=True))

    ps_sq = jnp.maximum(a2 + run, 1e-12)
    maxv = jnp.max(ps_sq)
    iota_p = lax.broadcasted_iota(jnp.int32, (_P, 1), 0)
    p_star = jnp.min(jnp.where(ps_sq == maxv, iota_p, _P))
    score = jnp.sqrt(maxv)

    feats = emb_ref[pl.ds(p_star, 1), :]
    a2p = jnp.sum(feats * feats)
    CS = cs_ref[...]
    b2row = b2_ref[...]
    iota_m = lax.broadcasted_iota(jnp.int32, (1, _M), 1)

    drow = b2row - 2.0 * _dot_nt(feats, CS)
    dmin = jnp.min(drow)
    nn_idx = jnp.min(jnp.where(drow == dmin, iota_m, _M))

    nn = cs_ref[pl.ds(nn_idx, 1), :]
    nn2 = jnp.sum(nn * nn)
    dv = jnp.sqrt(jnp.maximum(nn2 + b2row - 2.0 * _dot_nt(nn, CS), 1e-12))

    dks = []
    for _ in range(_K):
        mv = jnp.min(dv)
        ik = jnp.min(jnp.where(dv == mv, iota_m, _M))
        row = cs_ref[pl.ds(ik, 1), :]
        b2k = jnp.sum(row * row)
        dk = jnp.sqrt(jnp.maximum(a2p + b2k - 2.0 * jnp.sum(feats * row),
                                  1e-12))
        dks.append(dk)
        dv = jnp.where(iota_m == ik, jnp.inf, dv)

    m9 = dks[0]
    for d in dks[1:]:
        m9 = jnp.maximum(m9, d)
    es = [jnp.exp(d - m9) for d in dks]
    tot = es[0]
    for e in es[1:]:
        tot = tot + e
    w = 1.0 - es[0] / tot
    out_ref[...] = jnp.broadcast_to(w * score, (1, 1, 128))


def kernel(embedding, batch_size, embedding_coreset):
    res = pl.pallas_call(
        _body,
        grid=(_B,),
        in_specs=[
            pl.BlockSpec((_P, _D), lambda b: (b, 0)),
            pl.BlockSpec((_M, _D), lambda b: (0, 0)),
        ],
        out_specs=pl.BlockSpec((1, 1, 128), lambda b: (b, 0, 0)),
        out_shape=jax.ShapeDtypeStruct((_B, 1, 128), jnp.float32),
        scratch_shapes=[pltpu.VMEM((1, _M), jnp.float32)],
        compiler_params=pltpu.CompilerParams(
            dimension_semantics=("arbitrary",),
        ),
    )(embedding, embedding_coreset)
    return res[:, 0, 0] + 0.0 * batch_size


# speedup vs baseline: 3.2996x; 1.7440x over previous
import jax
import jax.numpy as jnp
from jax import lax
from jax.experimental import pallas as pl
from jax.experimental.pallas import tpu as pltpu

_B = 8
_P = 784
_M = 8192
_D = 384
_K = 9
_CHUNK = 2048


def _dot_nt(x, y):
    return lax.dot_general(x, y, (((1,), (1,)), ((), ())),
                           preferred_element_type=jnp.float32)


def _body(emb_ref, cs_ref, out_ref, b2h_ref, feats_ref, score_ref):
    b = pl.program_id(0)
    ones_row = jnp.ones((1, _D), jnp.float32)

    @pl.when(b == 0)
    def _():
        for c in range(_M // _CHUNK):
            Bc = cs_ref[c * _CHUNK:(c + 1) * _CHUNK, :]
            b2h_ref[:, c * _CHUNK:(c + 1) * _CHUNK] = 0.5 * _dot_nt(
                ones_row, Bc * Bc)

    A = emb_ref[...]
    a2 = jnp.sum(A * A, axis=1, keepdims=True)

    run128 = jnp.full((_P, 128), -jnp.inf, jnp.float32)
    for c in range(_M // _CHUNK):
        Bc = cs_ref[c * _CHUNK:(c + 1) * _CHUNK, :]
        H = _dot_nt(A, Bc) - b2h_ref[:, c * _CHUNK:(c + 1) * _CHUNK]
        part = H[:, 0:128]
        for l in range(1, _CHUNK // 128):
            part = jnp.maximum(part, H[:, l * 128:(l + 1) * 128])
        run128 = jnp.maximum(run128, part)

    hmax = jnp.max(run128, axis=1, keepdims=True)
    ps_sq = jnp.maximum(a2 - 2.0 * hmax, 1e-12)
    maxv = jnp.max(ps_sq)
    iota_p = lax.broadcasted_iota(jnp.int32, (_P, 1), 0)
    p_star = jnp.min(jnp.where(ps_sq == maxv, iota_p, _P))
    score = jnp.sqrt(maxv)

    feats = emb_ref[pl.ds(p_star, 1), :]
    feats_ref[pl.ds(b, 1), :, :] = feats[None, :, :]
    score_ref[pl.ds(b, 1), :, :] = jnp.broadcast_to(score, (1, 1, 128))

    @pl.when(b == _B - 1)
    def _():
        feats_all = jnp.concatenate([feats_ref[i] for i in range(_B)],
                                    axis=0)
        score_col = jnp.concatenate([score_ref[i] for i in range(_B)],
                                    axis=0)[:, 0:1]
        CS = cs_ref[...]
        b2row = 2.0 * b2h_ref[...]
        iota_m = lax.broadcasted_iota(jnp.int32, (_B, _M), 1)

        a2p = jnp.sum(feats_all * feats_all, axis=1, keepdims=True)
        Sf = b2row - 2.0 * _dot_nt(feats_all, CS)
        mf = jnp.min(Sf, axis=1, keepdims=True)
        nn_idx = jnp.min(jnp.where(Sf == mf, iota_m, _M),
                         axis=1, keepdims=True)
        Df = jnp.sqrt(jnp.maximum(a2p + Sf, 1e-12))

        nn_rows = []
        for i in range(_B):
            idx_i = jnp.sum(nn_idx[i:i + 1, 0:1])
            nn_rows.append(cs_ref[pl.ds(idx_i, 1), :])
        NN = jnp.concatenate(nn_rows, axis=0)
        nn2 = jnp.sum(NN * NN, axis=1, keepdims=True)
        dv = jnp.sqrt(jnp.maximum(nn2 + b2row - 2.0 * _dot_nt(NN, CS),
                                  1e-12))

        dks = []
        for _ in range(_K):
            mv = jnp.min(dv, axis=1, keepdims=True)
            ik = jnp.min(jnp.where(dv == mv, iota_m, _M),
                         axis=1, keepdims=True)
            sel = iota_m == ik
            dks.append(jnp.sum(jnp.where(sel, Df, 0.0), axis=1,
                               keepdims=True))
            dv = jnp.where(sel, jnp.inf, dv)

        m9 = dks[0]
        for d in dks[1:]:
            m9 = jnp.maximum(m9, d)
        es = [jnp.exp(d - m9) for d in dks]
        tot = es[0]
        for e in es[1:]:
            tot = tot + e
        w = 1.0 - es[0] / tot
        out_ref[...] = jnp.broadcast_to(w * score_col, (_B, 128))


def kernel(embedding, batch_size, embedding_coreset):
    res = pl.pallas_call(
        _body,
        grid=(_B,),
        in_specs=[
            pl.BlockSpec((_P, _D), lambda b: (b, 0)),
            pl.BlockSpec((_M, _D), lambda b: (0, 0)),
        ],
        out_specs=pl.BlockSpec((_B, 128), lambda b: (0, 0)),
        out_shape=jax.ShapeDtypeStruct((_B, 128), jnp.float32),
        scratch_shapes=[
            pltpu.VMEM((1, _M), jnp.float32),
            pltpu.VMEM((_B, 1, _D), jnp.float32),
            pltpu.VMEM((_B, 1, 128), jnp.float32),
        ],
        compiler_params=pltpu.CompilerParams(
            dimension_semantics=("arbitrary",),
        ),
    )(embedding, embedding_coreset)
    return res[:, 0] + 0.0 * batch_size
